# Initial kernel scaffold; baseline (speedup 1.0000x reference)
import jax
import jax.numpy as jnp
from jax.experimental import pallas as pl


def kernel(x, timestep, W):
    raise NotImplementedError("write your pallas kernel here")


# speedup vs baseline: 3.1015x; 3.1015x over previous
import jax
import jax.numpy as jnp
from jax.experimental import pallas as pl
from jax.experimental.pallas import tpu as pltpu

_BLK = 1024


def _body(ts_ref, x_ref, w_ref, o_ref):
    t = ts_ref[0]
    row = w_ref[t, :]
    o_ref[...] = x_ref[...] + row[None, :]


def kernel(x, timestep, W):
    B, D = x.shape
    ts = jnp.asarray(timestep, dtype=jnp.int32).reshape((1,))
    grid = (B // _BLK,)
    return pl.pallas_call(
        _body,
        grid_spec=pltpu.PrefetchScalarGridSpec(
            num_scalar_prefetch=1,
            grid=grid,
            in_specs=[
                pl.BlockSpec((_BLK, D), lambda i, ts: (i, 0)),
                pl.BlockSpec(W.shape, lambda i, ts: (0, 0)),
            ],
            out_specs=pl.BlockSpec((_BLK, D), lambda i, ts: (i, 0)),
        ),
        out_shape=jax.ShapeDtypeStruct((B, D), x.dtype),
    )(ts, x, W)


# speedup vs baseline: 3.2144x; 1.0364x over previous
import jax
import jax.numpy as jnp
from jax.experimental import pallas as pl
from jax.experimental.pallas import tpu as pltpu

_BLK = 2048


def _body(ts_ref, x_ref, w_ref, o_ref):
    t = ts_ref[0]
    row = w_ref[t, :]
    o_ref[...] = x_ref[...] + row[None, :]


def kernel(x, timestep, W):
    B, D = x.shape
    ts = jnp.asarray(timestep, dtype=jnp.int32).reshape((1,))
    grid = (B // _BLK,)
    return pl.pallas_call(
        _body,
        grid_spec=pltpu.PrefetchScalarGridSpec(
            num_scalar_prefetch=1,
            grid=grid,
            in_specs=[
                pl.BlockSpec((_BLK, D), lambda i, ts: (i, 0)),
                pl.BlockSpec(W.shape, lambda i, ts: (0, 0)),
            ],
            out_specs=pl.BlockSpec((_BLK, D), lambda i, ts: (i, 0)),
        ),
        out_shape=jax.ShapeDtypeStruct((B, D), x.dtype),
    )(ts, x, W)
